# Initial kernel scaffold; baseline (speedup 1.0000x reference)
import jax
import jax.numpy as jnp
from jax.experimental import pallas as pl


def kernel(v, a, psi_w0, psi_b0, psi_w1, psi_b1, psi_w2, psi_b2, psi_w3, psi_b3, phi_w0, phi_b0, phi_w1, phi_b1, phi_w2, phi_b2, phi_w3, phi_b3):
    raise NotImplementedError("write your pallas kernel here")


# speedup vs baseline: 2.9563x; 2.9563x over previous
import functools

import jax
import jax.numpy as jnp
from jax.experimental import pallas as pl
from jax.experimental.pallas import tpu as pltpu

TILE_M = 2048


def _round_up(x, m):
    return (x + m - 1) // m * m


def _mlp3(x, w1, b1, w2, b2, w3, b3):
    h = jnp.dot(x, w1, preferred_element_type=jnp.float32) + b1
    h = jnp.maximum(h, 0.0)
    h = jnp.dot(h, w2, preferred_element_type=jnp.float32) + b2
    h = jnp.maximum(h, 0.0)
    h = jnp.dot(h, w3, preferred_element_type=jnp.float32) + b3
    return jnp.maximum(h, 0.0)


def _psi_kernel(a_ref, v_ref, w1, b1, w2, b2, w3, b3, w4, b4, u_ref):
    nt = pl.program_id(1)

    @pl.when(nt == 0)
    def _():
        u_ref[...] = jnp.zeros_like(u_ref)

    h = _mlp3(a_ref[0], w1[...], b1[...], w2[...], b2[...], w3[...], b3[...])
    psi = jnp.dot(h, w4[...], preferred_element_type=jnp.float32) + b4[...]
    u_ref[0] += jax.lax.dot_general(
        psi, v_ref[0], (((0,), (0,)), ((), ())),
        preferred_element_type=jnp.float32)


def _phi_kernel(a_ref, u_ref, w1, b1, w2, b2, w3, b3, w4, b4, o_ref, *,
                n_inv):
    h = _mlp3(a_ref[0], w1[...], b1[...], w2[...], b2[...], w3[...], b3[...])
    u = u_ref[0]
    d_dim, o_dim = u.shape[0], o_ref.shape[-1]
    blk = jax.lax.broadcasted_iota(jnp.int32, (d_dim, o_dim), 0) // (
        d_dim // o_dim)
    oix = jax.lax.broadcasted_iota(jnp.int32, (d_dim, o_dim), 1)
    su = jnp.where(blk == oix, u * n_inv, 0.0)
    w4_eff = jnp.dot(w4[...], su, preferred_element_type=jnp.float32)
    b4_eff = jnp.dot(b4[...], su, preferred_element_type=jnp.float32)
    out = jnp.dot(h, w4_eff, preferred_element_type=jnp.float32) + b4_eff
    o_ref[0] = out.astype(o_ref.dtype)


def _full_spec(p):
    return pl.BlockSpec(p.shape, lambda b, nt: (0, 0))


def kernel(v, a, psi_w0, psi_b0, psi_w1, psi_b1, psi_w2, psi_b2, psi_w3,
           psi_b3, phi_w0, phi_b0, phi_w1, phi_b1, phi_w2, phi_b2, phi_w3,
           phi_b3):
    B, N, I = v.shape
    D = psi_w3.shape[1]
    O = I
    R = D // (O * I)

    tile_m = min(TILE_M, _round_up(N, 8))
    n_pad = _round_up(N, tile_m)
    n_tiles = n_pad // tile_m
    if n_pad != N:
        a_p = jnp.pad(a, ((0, 0), (0, n_pad - N), (0, 0)))
        v_p = jnp.pad(v, ((0, 0), (0, n_pad - N), (0, 0)))
    else:
        a_p, v_p = a, v

    psi_flat = [psi_w0, psi_b0, psi_w1, psi_b1, psi_w2, psi_b2, psi_w3,
                psi_b3]
    phi_flat = [phi_w0, phi_b0, phi_w1, phi_b1, phi_w2, phi_b2, phi_w3,
                phi_b3]

    u_dt = pl.pallas_call(
        _psi_kernel,
        grid=(B, n_tiles),
        in_specs=[pl.BlockSpec((1, tile_m, 3), lambda b, nt: (b, nt, 0)),
                  pl.BlockSpec((1, tile_m, I), lambda b, nt: (b, nt, 0))]
                 + [_full_spec(p) for p in psi_flat],
        out_specs=pl.BlockSpec((1, D, I), lambda b, nt: (b, 0, 0)),
        out_shape=jax.ShapeDtypeStruct((B, D, I), jnp.float32),
        compiler_params=pltpu.CompilerParams(
            dimension_semantics=("parallel", "arbitrary")),
    )(a_p, v_p, *psi_flat)

    imap = (jnp.arange(D) % (I * R)) // R
    u = jnp.take_along_axis(u_dt, imap[None, :, None], axis=2)

    out_pad = pl.pallas_call(
        functools.partial(_phi_kernel, n_inv=1.0 / float(N)),
        grid=(B, n_tiles),
        in_specs=[pl.BlockSpec((1, tile_m, 3), lambda b, nt: (b, nt, 0)),
                  pl.BlockSpec((1, D, 1), lambda b, nt: (b, 0, 0))]
                 + [_full_spec(p) for p in phi_flat],
        out_specs=pl.BlockSpec((1, tile_m, O), lambda b, nt: (b, nt, 0)),
        out_shape=jax.ShapeDtypeStruct((B, n_pad, O), v.dtype),
        compiler_params=pltpu.CompilerParams(
            dimension_semantics=("parallel", "parallel")),
    )(a_p, u, *phi_flat)

    return out_pad[:, :N, :]


# speedup vs baseline: 3.4471x; 1.1660x over previous
import functools

import jax
import jax.numpy as jnp
from jax.experimental import pallas as pl
from jax.experimental.pallas import tpu as pltpu

TILE_M = 4096


def _round_up(x, m):
    return (x + m - 1) // m * m


def _mlp3(x, w1, b1, w2, b2, w3, b3):
    h = jnp.dot(x, w1, preferred_element_type=jnp.float32) + b1
    h = jnp.maximum(h, 0.0)
    h = jnp.dot(h, w2, preferred_element_type=jnp.float32) + b2
    h = jnp.maximum(h, 0.0)
    h = jnp.dot(h, w3, preferred_element_type=jnp.float32) + b3
    return jnp.maximum(h, 0.0)


def _psi_kernel(a_ref, v_ref, w1, b1, w2, b2, w3, b3, w4, b4, u_ref):
    nt = pl.program_id(1)

    @pl.when(nt == 0)
    def _():
        u_ref[...] = jnp.zeros_like(u_ref)

    h = _mlp3(a_ref[0], w1[...], b1[...], w2[...], b2[...], w3[...], b3[...])
    psi = jnp.dot(h, w4[...], preferred_element_type=jnp.float32) + b4[...]
    u_ref[0] += jax.lax.dot_general(
        psi, v_ref[0], (((0,), (0,)), ((), ())),
        preferred_element_type=jnp.float32)


def _phi_kernel(a_ref, u_ref, w1, b1, w2, b2, w3, b3, w4, b4, o_ref, *,
                n_inv):
    h = _mlp3(a_ref[0], w1[...], b1[...], w2[...], b2[...], w3[...], b3[...])
    u = u_ref[0]
    d_dim, o_dim = u.shape[0], o_ref.shape[-1]
    blk = jax.lax.broadcasted_iota(jnp.int32, (d_dim, o_dim), 0) // (
        d_dim // o_dim)
    oix = jax.lax.broadcasted_iota(jnp.int32, (d_dim, o_dim), 1)
    su = jnp.where(blk == oix, u * n_inv, 0.0)
    w4_eff = jnp.dot(w4[...], su, preferred_element_type=jnp.float32)
    b4_eff = jnp.dot(b4[...], su, preferred_element_type=jnp.float32)
    out = jnp.dot(h, w4_eff, preferred_element_type=jnp.float32) + b4_eff
    o_ref[0] = out.astype(o_ref.dtype)


def _full_spec(p):
    return pl.BlockSpec(p.shape, lambda b, nt: (0, 0))


def kernel(v, a, psi_w0, psi_b0, psi_w1, psi_b1, psi_w2, psi_b2, psi_w3,
           psi_b3, phi_w0, phi_b0, phi_w1, phi_b1, phi_w2, phi_b2, phi_w3,
           phi_b3):
    B, N, I = v.shape
    D = psi_w3.shape[1]
    O = I
    R = D // (O * I)

    tile_m = min(TILE_M, _round_up(N, 8))
    n_pad = _round_up(N, tile_m)
    n_tiles = n_pad // tile_m
    if n_pad != N:
        a_p = jnp.pad(a, ((0, 0), (0, n_pad - N), (0, 0)))
        v_p = jnp.pad(v, ((0, 0), (0, n_pad - N), (0, 0)))
    else:
        a_p, v_p = a, v

    psi_flat = [psi_w0, psi_b0, psi_w1, psi_b1, psi_w2, psi_b2, psi_w3,
                psi_b3]
    phi_flat = [phi_w0, phi_b0, phi_w1, phi_b1, phi_w2, phi_b2, phi_w3,
                phi_b3]

    u_dt = pl.pallas_call(
        _psi_kernel,
        grid=(B, n_tiles),
        in_specs=[pl.BlockSpec((1, tile_m, 3), lambda b, nt: (b, nt, 0)),
                  pl.BlockSpec((1, tile_m, I), lambda b, nt: (b, nt, 0))]
                 + [_full_spec(p) for p in psi_flat],
        out_specs=pl.BlockSpec((1, D, I), lambda b, nt: (b, 0, 0)),
        out_shape=jax.ShapeDtypeStruct((B, D, I), jnp.float32),
        compiler_params=pltpu.CompilerParams(
            dimension_semantics=("parallel", "arbitrary")),
    )(a_p, v_p, *psi_flat)

    imap = (jnp.arange(D) % (I * R)) // R
    u = jnp.take_along_axis(u_dt, imap[None, :, None], axis=2)

    out_pad = pl.pallas_call(
        functools.partial(_phi_kernel, n_inv=1.0 / float(N)),
        grid=(B, n_tiles),
        in_specs=[pl.BlockSpec((1, tile_m, 3), lambda b, nt: (b, nt, 0)),
                  pl.BlockSpec((1, D, 1), lambda b, nt: (b, 0, 0))]
                 + [_full_spec(p) for p in phi_flat],
        out_specs=pl.BlockSpec((1, tile_m, O), lambda b, nt: (b, nt, 0)),
        out_shape=jax.ShapeDtypeStruct((B, n_pad, O), v.dtype),
        compiler_params=pltpu.CompilerParams(
            dimension_semantics=("parallel", "parallel")),
    )(a_p, u, *phi_flat)

    return out_pad[:, :N, :]
